# baseline (speedup 1.0000x reference)
import functools

import jax
import jax.numpy as jnp
from jax import lax
from jax.experimental import pallas as pl
from jax.experimental.pallas import tpu as pltpu
from jax.experimental.pallas import tpu_sc as plsc

_NC, _NS = 2, 16
_NW = _NC * _NS
_R = 8
_L = 16
_SC_BATCH = 3
_BS = 2048


def _sc_body(S, D, x_hbm, t_hbm, pos_hbm, out_hbm, *refs):
    bufx = refs[0:2]
    buft = refs[2:4]
    idx_all = refs[4]
    sld = refs[5:7]
    sst = refs[7:9]
    stb = refs[9:11]

    cid = lax.axis_index("c")
    sid = lax.axis_index("s")
    wid = sid * _NC + cid
    rows_per_w = S // _NW
    nchunks = rows_per_w // _R
    seqbase = wid * rows_per_w

    def x_slice(ci):
        return x_hbm.at[_SC_BATCH, pl.ds(seqbase + ci * _R, _R)]

    def out_slice(ci):
        return out_hbm.at[pl.ds(seqbase + ci * _R, _R)]

    def start_gather(ci, p):
        idx = idx_all.at[pl.ds(ci * _R, _R)]
        pltpu.async_copy(t_hbm.at[idx], buft[p], stb[p])

    pltpu.sync_copy(pos_hbm.at[pl.ds(seqbase, rows_per_w)], idx_all)

    start_gather(0, 0)
    pltpu.async_copy(x_slice(0), bufx[0], sld[0])

    def two_chunks(i, carry):
        for p in range(2):
            ci = 2 * i + p
            q = 1 - p
            pltpu.make_async_copy(
                t_hbm.at[idx_all.at[pl.ds(0, _R)]], buft[p], stb[p]).wait()
            pltpu.make_async_copy(x_slice(0), bufx[p], sld[p]).wait()

            @pl.when(ci + 1 < nchunks)
            def _():
                start_gather(ci + 1, q)
                @pl.when(ci >= 1)
                def _():
                    pltpu.make_async_copy(
                        bufx[q], out_slice(0), sst[q]).wait()
                pltpu.async_copy(x_slice(ci + 1), bufx[q], sld[q])

            @plsc.parallel_loop(0, D // _L, unroll=2)
            def _(j):
                base = j * _L
                for r in range(_R):
                    bufx[p][r, pl.ds(base, _L)] = (
                        bufx[p][r, pl.ds(base, _L)]
                        + buft[p][r, pl.ds(base, _L)]
                    )

            pltpu.async_copy(bufx[p], out_slice(ci), sst[p])
        return carry

    lax.fori_loop(0, nchunks // 2, two_chunks, 0)

    for p in range(2):
        pltpu.make_async_copy(bufx[p], out_slice(0), sst[p]).wait()


def _sc_part(x, table, pos):
    B, S, D = x.shape
    mesh = plsc.VectorSubcoreMesh(core_axis_name="c", subcore_axis_name="s")
    body = functools.partial(_sc_body, S, D)
    f = pl.kernel(
        body,
        out_type=jax.ShapeDtypeStruct((S, D), x.dtype),
        mesh=mesh,
        scratch_types=(
            [pltpu.VMEM((_R, D), jnp.float32) for _ in range(4)]
            + [pltpu.VMEM((S // _NW,), jnp.int32)]
            + [pltpu.SemaphoreType.DMA for _ in range(6)]
        ),
    )
    return f(x, table, pos)


def _tc_body(off_ref, x_ref, t_ref, o_ref):
    del off_ref
    o_ref[...] = x_ref[...] + t_ref[...]


def _tc_part(x, table, off):
    B, S, D = x.shape
    nb = B - 1
    spec = pltpu.PrefetchScalarGridSpec(
        num_scalar_prefetch=1,
        grid=(S // _BS, nb),
        in_specs=[
            pl.BlockSpec((1, _BS, D), lambda i, j, off: (j, i, 0)),
            pl.BlockSpec((_BS, D), lambda i, j, off: (i + off[0] // _BS, 0)),
        ],
        out_specs=pl.BlockSpec((1, _BS, D), lambda i, j, off: (j, i, 0)),
    )
    return pl.pallas_call(
        _tc_body,
        grid_spec=spec,
        out_shape=jax.ShapeDtypeStruct((nb, S, D), x.dtype),
        compiler_params=pltpu.CompilerParams(
            dimension_semantics=("arbitrary", "arbitrary"),
        ),
    )(off, x, table)


def kernel(x, table, offset=0):
    off = jnp.asarray(offset, jnp.int32).reshape((1,))
    S = x.shape[1]
    pos = off[0] + lax.iota(jnp.int32, S)
    sc_out = _sc_part(x, table, pos)
    tc_out = _tc_part(x, table, off)
    return jnp.concatenate([tc_out, sc_out[None]], axis=0)


# speedup vs baseline: 1.5985x; 1.5985x over previous
import functools

import jax
import jax.numpy as jnp
from jax import lax
from jax.experimental import pallas as pl
from jax.experimental.pallas import tpu as pltpu
from jax.experimental.pallas import tpu_sc as plsc

_NC, _NS = 2, 16
_NW = _NC * _NS
_R = 8
_L = 16


def _sc_body(B, S, D, x_hbm, t_hbm, pos_hbm, xrow_hbm, out_hbm, *refs):
    bufx = refs[0:2]
    buft = refs[2:4]
    idx_all = refs[4]
    xidx = refs[5]
    sld = refs[6:8]
    sst = refs[8:10]
    stb = refs[10:12]

    cid = lax.axis_index("c")
    sid = lax.axis_index("s")
    wid = sid * _NC + cid
    rows_per_w = S // _NW
    nchunks = rows_per_w // _R
    seqbase = wid * rows_per_w

    def start_gather(ci, p):
        idx = idx_all.at[pl.ds(ci * _R, _R)]
        pltpu.async_copy(t_hbm.at[idx], buft[p], stb[p])

    def start_xload(ci, p):
        pltpu.async_copy(x_hbm.at[xidx.at[ci]], bufx[p], sld[p])

    pltpu.sync_copy(pos_hbm.at[pl.ds(seqbase, rows_per_w)], idx_all)
    pltpu.sync_copy(xrow_hbm.at[wid], xidx)

    start_gather(0, 0)
    start_xload(0, 0)

    def two_chunks(i, carry):
        for p in range(2):
            ci = 2 * i + p
            q = 1 - p
            pltpu.make_async_copy(
                t_hbm.at[idx_all.at[pl.ds(0, _R)]], buft[p], stb[p]).wait()
            pltpu.make_async_copy(
                x_hbm.at[xidx.at[0]], bufx[p], sld[p]).wait()

            @pl.when(ci + 1 < nchunks)
            def _():
                start_gather(ci + 1, q)
                @pl.when(ci >= 1)
                def _():
                    pltpu.make_async_copy(
                        bufx[q], out_hbm.at[xidx.at[0]], sst[q]).wait()
                start_xload(ci + 1, q)

            @plsc.parallel_loop(0, D // _L, unroll=2)
            def _(j):
                base = j * _L
                for r in range(_R):
                    vt = buft[p][r, pl.ds(base, _L)]
                    for b in range(B):
                        bufx[p][b * _R + r, pl.ds(base, _L)] = (
                            bufx[p][b * _R + r, pl.ds(base, _L)] + vt
                        )

            pltpu.async_copy(bufx[p], out_hbm.at[xidx.at[ci]], sst[p])
        return carry

    lax.fori_loop(0, nchunks // 2, two_chunks, 0)

    for p in range(2):
        pltpu.make_async_copy(bufx[p], out_hbm.at[xidx.at[0]], sst[p]).wait()


def kernel(x, table, offset=0):
    B, S, D = x.shape
    off = jnp.asarray(offset, jnp.int32)
    pos = off + lax.iota(jnp.int32, S)
    rows_per_w = S // _NW
    nchunks = rows_per_w // _R
    w_ids = lax.broadcasted_iota(jnp.int32, (_NW, nchunks, B, _R), 0)
    c_ids = lax.broadcasted_iota(jnp.int32, (_NW, nchunks, B, _R), 1)
    b_ids = lax.broadcasted_iota(jnp.int32, (_NW, nchunks, B, _R), 2)
    r_ids = lax.broadcasted_iota(jnp.int32, (_NW, nchunks, B, _R), 3)
    xrow = (b_ids * S + w_ids * rows_per_w + c_ids * _R + r_ids).reshape(
        _NW, nchunks, B * _R)

    mesh = plsc.VectorSubcoreMesh(core_axis_name="c", subcore_axis_name="s")
    body = functools.partial(_sc_body, B, S, D)
    f = pl.kernel(
        body,
        out_type=jax.ShapeDtypeStruct((B * S, D), x.dtype),
        mesh=mesh,
        scratch_types=(
            [pltpu.VMEM((B * _R, D), jnp.float32) for _ in range(2)]
            + [pltpu.VMEM((_R, D), jnp.float32) for _ in range(2)]
            + [pltpu.VMEM((rows_per_w,), jnp.int32)]
            + [pltpu.VMEM((nchunks, B * _R), jnp.int32)]
            + [pltpu.SemaphoreType.DMA for _ in range(6)]
        ),
    )
    out = f(x.reshape(B * S, D), table, pos, xrow)
    return out.reshape(B, S, D)


# speedup vs baseline: 1.6219x; 1.0147x over previous
import functools

import jax
import jax.numpy as jnp
from jax import lax
from jax.experimental import pallas as pl
from jax.experimental.pallas import tpu as pltpu
from jax.experimental.pallas import tpu_sc as plsc

_NC, _NS = 2, 16
_NW = _NC * _NS
_R = 8
_L = 16


def _sc_body(B, S, D, x_hbm, t_hbm, pos_hbm, out_hbm, *refs):
    bufx = (refs[0:4], refs[4:8])
    buft = refs[8:10]
    idx_all = refs[10]
    sld = (refs[11:15], refs[15:19])
    sst = (refs[19:23], refs[23:27])
    stb = refs[27:29]

    cid = lax.axis_index("c")
    sid = lax.axis_index("s")
    wid = sid * _NC + cid
    rows_per_w = S // _NW
    nchunks = rows_per_w // _R
    seqbase = wid * rows_per_w

    def x_slice(ci, b):
        return x_hbm.at[b, pl.ds(seqbase + ci * _R, _R)]

    def out_slice(ci, b):
        return out_hbm.at[b, pl.ds(seqbase + ci * _R, _R)]

    def start_gather(ci, p):
        idx = idx_all.at[pl.ds(ci * _R, _R)]
        pltpu.async_copy(t_hbm.at[idx], buft[p], stb[p])

    pltpu.sync_copy(pos_hbm.at[pl.ds(seqbase, rows_per_w)], idx_all)

    start_gather(0, 0)
    for b in range(B):
        pltpu.async_copy(x_slice(0, b), bufx[0][b], sld[0][b])

    def two_chunks(i, carry):
        for p in range(2):
            ci = 2 * i + p
            q = 1 - p
            pltpu.make_async_copy(
                t_hbm.at[idx_all.at[pl.ds(0, _R)]], buft[p], stb[p]).wait()
            @pl.when(ci + 1 < nchunks)
            def _():
                start_gather(ci + 1, q)

            for b in range(B):
                pltpu.make_async_copy(
                    x_slice(0, 0), bufx[p][b], sld[p][b]).wait()
            @pl.when(ci + 1 < nchunks)
            def _():
                for b in range(B):
                    @pl.when(ci >= 1)
                    def _():
                        pltpu.make_async_copy(
                            bufx[q][b], out_slice(0, 0), sst[q][b]).wait()
                    pltpu.async_copy(x_slice(ci + 1, b), bufx[q][b], sld[q][b])

            @plsc.parallel_loop(0, D // _L, unroll=2)
            def _(j):
                base = j * _L
                for r in range(_R):
                    vt = buft[p][r, pl.ds(base, _L)]
                    for b in range(B):
                        bufx[p][b][r, pl.ds(base, _L)] = (
                            bufx[p][b][r, pl.ds(base, _L)] + vt
                        )

            for b in range(B):
                pltpu.async_copy(bufx[p][b], out_slice(ci, b), sst[p][b])
        return carry

    lax.fori_loop(0, nchunks // 2, two_chunks, 0)

    for p in range(2):
        for b in range(B):
            pltpu.make_async_copy(bufx[p][b], out_slice(0, 0), sst[p][b]).wait()


def kernel(x, table, offset=0):
    B, S, D = x.shape
    pos = jnp.asarray(offset, jnp.int32) + lax.iota(jnp.int32, S)
    mesh = plsc.VectorSubcoreMesh(core_axis_name="c", subcore_axis_name="s")
    body = functools.partial(_sc_body, B, S, D)
    f = pl.kernel(
        body,
        out_type=jax.ShapeDtypeStruct((B, S, D), x.dtype),
        mesh=mesh,
        scratch_types=(
            [pltpu.VMEM((_R, D), jnp.float32) for _ in range(8)]
            + [pltpu.VMEM((_R, D), jnp.float32) for _ in range(2)]
            + [pltpu.VMEM((S // _NW,), jnp.int32)]
            + [pltpu.SemaphoreType.DMA for _ in range(18)]
        ),
    )
    return f(x, table, pos)


# speedup vs baseline: 1.6331x; 1.0069x over previous
import functools

import jax
import jax.numpy as jnp
from jax import lax
from jax.experimental import pallas as pl
from jax.experimental.pallas import tpu as pltpu
from jax.experimental.pallas import tpu_sc as plsc

_NC, _NS = 2, 16
_NW = _NC * _NS
_R = 8
_L = 16
_P = 3


def _sc_body(B, S, D, x_hbm, t_hbm, pos_hbm, out_hbm, *refs):
    bufx = tuple(refs[4 * p:4 * p + 4] for p in range(_P))
    buft = refs[12:15]
    idx_all = refs[15]
    sld = tuple(refs[16 + 4 * p:20 + 4 * p] for p in range(_P))
    sst = tuple(refs[28 + 4 * p:32 + 4 * p] for p in range(_P))
    stb = refs[40:43]

    cid = lax.axis_index("c")
    sid = lax.axis_index("s")
    wid = sid * _NC + cid
    rows_per_w = S // _NW
    nchunks = rows_per_w // _R
    seqbase = wid * rows_per_w

    def x_slice(ci, b):
        return x_hbm.at[b, pl.ds(seqbase + ci * _R, _R)]

    def out_slice(ci, b):
        return out_hbm.at[b, pl.ds(seqbase + ci * _R, _R)]

    def start_chunk(ci, p):
        idx = idx_all.at[pl.ds(ci * _R, _R)]
        pltpu.async_copy(t_hbm.at[idx], buft[p], stb[p])
        for b in range(B):
            pltpu.async_copy(x_slice(ci, b), bufx[p][b], sld[p][b])

    def wait_stores(p):
        for b in range(B):
            pltpu.make_async_copy(bufx[p][b], out_slice(0, 0), sst[p][b]).wait()

    def process(ci, p, prefetch, storewait):
        pltpu.make_async_copy(
            t_hbm.at[idx_all.at[pl.ds(0, _R)]], buft[p], stb[p]).wait()
        for b in range(B):
            pltpu.make_async_copy(
                x_slice(0, 0), bufx[p][b], sld[p][b]).wait()

        tp = (p + 2) % _P
        if prefetch:
            if storewait:
                @pl.when(ci >= 1)
                def _():
                    wait_stores(tp)
                start_chunk(ci + 2, tp)
            else:
                wait_stores(tp)
                start_chunk(ci + 2, tp)

        @plsc.parallel_loop(0, D // _L, unroll=2)
        def _(j):
            base = j * _L
            for r in range(_R):
                vt = buft[p][r, pl.ds(base, _L)]
                for b in range(B):
                    bufx[p][b][r, pl.ds(base, _L)] = (
                        bufx[p][b][r, pl.ds(base, _L)] + vt
                    )

        for b in range(B):
            pltpu.async_copy(bufx[p][b], out_slice(ci, b), sst[p][b])

    pltpu.sync_copy(pos_hbm.at[pl.ds(seqbase, rows_per_w)], idx_all)

    start_chunk(0, 0)
    start_chunk(1, 1)

    def three_chunks(i, carry):
        ci = _P * i
        process(ci, 0, prefetch=True, storewait=True)
        process(ci + 1, 1, prefetch=True, storewait=True)
        process(ci + 2, 2, prefetch=True, storewait=True)
        return carry

    lax.fori_loop(0, (nchunks - 2) // _P, three_chunks, 0)

    process(nchunks - 2, (nchunks - 2) % _P, prefetch=False, storewait=False)
    process(nchunks - 1, (nchunks - 1) % _P, prefetch=False, storewait=False)
    for p in range(_P):
        wait_stores(p)


def kernel(x, table, offset=0):
    B, S, D = x.shape
    pos = jnp.asarray(offset, jnp.int32) + lax.iota(jnp.int32, S)
    mesh = plsc.VectorSubcoreMesh(core_axis_name="c", subcore_axis_name="s")
    body = functools.partial(_sc_body, B, S, D)
    f = pl.kernel(
        body,
        out_type=jax.ShapeDtypeStruct((B, S, D), x.dtype),
        mesh=mesh,
        scratch_types=(
            [pltpu.VMEM((_R, D), jnp.float32) for _ in range(12)]
            + [pltpu.VMEM((_R, D), jnp.float32) for _ in range(3)]
            + [pltpu.VMEM((S // _NW,), jnp.int32)]
            + [pltpu.SemaphoreType.DMA for _ in range(27)]
        ),
    )
    return f(x, table, pos)
